# Initial kernel scaffold; baseline (speedup 1.0000x reference)
import jax
import jax.numpy as jnp
from jax.experimental import pallas as pl


def kernel(x, W_router, W1, W2):
    raise NotImplementedError("write your pallas kernel here")


# speedup vs baseline: 1.1120x; 1.1120x over previous
import functools

import jax
import jax.numpy as jnp
from jax.experimental import pallas as pl
from jax.experimental.pallas import tpu as pltpu

_T, _D, _E, _H, _K = 2048, 1024, 8, 4096, 2
_CAP = 30.0
_LB = 0.01
_B = 128
_A = _T * _K
_NB = _A // _B + _E
_P = _NB * _B


def _router_body(x_ref, wr_ref, idx0_ref, idx1_ref, g0_ref, g1_ref, aux_ref):
    x = x_ref[...]
    wr = wr_ref[...]
    logits = jnp.dot(x, wr, preferred_element_type=jnp.float32)
    logits = _CAP * jnp.tanh(logits / _CAP)
    m = jnp.max(logits, axis=1, keepdims=True)
    p = jnp.exp(logits - m)
    probs = p / jnp.sum(p, axis=1, keepdims=True)
    cols = jax.lax.broadcasted_iota(jnp.int32, (_T, _E), 1)
    v0 = jnp.max(probs, axis=1, keepdims=True)
    a0 = jnp.min(jnp.where(probs == v0, cols, _E), axis=1, keepdims=True)
    probs_m = jnp.where(cols == a0, -1.0, probs)
    v1 = jnp.max(probs_m, axis=1, keepdims=True)
    a1 = jnp.min(jnp.where(probs_m == v1, cols, _E), axis=1, keepdims=True)
    s = v0 + v1 + 1e-9
    g0 = v0 / s
    g1 = v1 / s
    idx0_ref[...] = a0
    idx1_ref[...] = a1
    g0_ref[...] = g0
    g1_ref[...] = g1
    me = jnp.mean(probs, axis=0, keepdims=True)
    oh0 = jnp.where(cols == a0, g0, 0.0)
    oh1 = jnp.where(cols == a1, g1, 0.0)
    ce = jnp.sum(oh0 + oh1, axis=0, keepdims=True) / _T
    aux_ref[...] = _LB * _E * _K * jnp.sum(me * ce, keepdims=True)


def _route(x, W_router):
    return pl.pallas_call(
        _router_body,
        out_shape=(
            jax.ShapeDtypeStruct((_T, 1), jnp.int32),
            jax.ShapeDtypeStruct((_T, 1), jnp.int32),
            jax.ShapeDtypeStruct((_T, 1), jnp.float32),
            jax.ShapeDtypeStruct((_T, 1), jnp.float32),
            jax.ShapeDtypeStruct((1, 1), jnp.float32),
        ),
    )(x, W_router)


def _ffn_body(be_ref, xs_ref, w1_ref, w2_ref, ys_ref):
    xb = xs_ref[...].astype(jnp.bfloat16)
    h = jnp.dot(xb, w1_ref[0], preferred_element_type=jnp.float32)
    h = jnp.maximum(h, 0.0)
    h = (h * h).astype(jnp.bfloat16)
    ys_ref[...] = jnp.dot(h, w2_ref[0], preferred_element_type=jnp.float32)


def _grouped_ffn(block_expert, xs, w1b, w2b):
    grid_spec = pltpu.PrefetchScalarGridSpec(
        num_scalar_prefetch=1,
        grid=(_NB,),
        in_specs=[
            pl.BlockSpec((_B, _D), lambda i, be: (i, 0)),
            pl.BlockSpec((1, _D, _H), lambda i, be: (be[i], 0, 0)),
            pl.BlockSpec((1, _H, _D), lambda i, be: (be[i], 0, 0)),
        ],
        out_specs=pl.BlockSpec((_B, _D), lambda i, be: (i, 0)),
    )
    return pl.pallas_call(
        _ffn_body,
        grid_spec=grid_spec,
        out_shape=jax.ShapeDtypeStruct((_P, _D), jnp.float32),
    )(block_expert, xs, w1b, w2b)


def kernel(x, W_router, W1, W2):
    idx0, idx1, g0, g1, aux = _route(x, W_router)
    idx0 = idx0[:, 0]
    idx1 = idx1[:, 0]

    idx_all = jnp.concatenate([idx0, idx1])
    oh = (idx_all[:, None] == jnp.arange(_E)[None, :]).astype(jnp.int32)
    csum = jnp.cumsum(oh, axis=0)
    rank = jnp.sum((csum - oh) * oh, axis=1)
    counts = csum[-1]
    nblk = (counts + _B - 1) // _B
    bstart = jnp.concatenate(
        [jnp.zeros((1,), nblk.dtype), jnp.cumsum(nblk)[:-1]])
    pos = (bstart[idx_all] * _B + rank).astype(jnp.int32)
    bids = jnp.arange(_NB)
    block_expert = (jnp.sum(bids[:, None] >= bstart[None, :], axis=1)
                    .astype(jnp.int32) - 1)

    xs = jnp.zeros((_P, _D), jnp.float32).at[pos].set(
        jnp.concatenate([x, x], axis=0))

    ys = _grouped_ffn(block_expert, xs,
                      W1.astype(jnp.bfloat16), W2.astype(jnp.bfloat16))

    y = g0 * ys[pos[:_T]] + g1 * ys[pos[_T:]]
    return y, aux[0, 0]
